# Initial kernel scaffold; baseline (speedup 1.0000x reference)
import jax
import jax.numpy as jnp
from jax.experimental import pallas as pl


def kernel(source_nodes, source_node_raw_features, timestamps, n_layers, memory, time_W, time_b):
    raise NotImplementedError("write your pallas kernel here")


# speedup vs baseline: 2.1749x; 2.1749x over previous
import functools

import jax
import jax.numpy as jnp
from jax import lax
from jax.experimental import pallas as pl
from jax.experimental.pallas import tpu as pltpu
from jax.experimental.pallas import tpu_sc as plsc

N_NODES = 100000
B = 625000
D = 128
LANES = 16

NC = 2
NS = 16
NW = NC * NS

C = 128

GROUPS = B // 8
GPW_BASE = GROUPS // NW
GPW_REM = GROUPS - GPW_BASE * NW
N_MAX = 8 * (GPW_BASE + 1)
N_CHUNKS = -(-N_MAX // C)


def _sc_body(idx_hbm, feat_hbm, mem_hbm, out_hbm, idx_v, rows_v, feat_v, sem):
    wid = lax.axis_index("s") * NC + lax.axis_index("c")
    extra = jnp.minimum(wid, GPW_REM)
    start = 8 * (wid * GPW_BASE + extra)
    n_rows = 8 * (GPW_BASE + jnp.where(wid < GPW_REM, 1, 0))

    def chunk(i, _):
        base = start + jnp.minimum(i * C, n_rows - C)
        pltpu.sync_copy(idx_hbm.at[pl.ds(base, C)], idx_v)
        gather = pltpu.async_copy(mem_hbm.at[idx_v], rows_v, sem)
        pltpu.sync_copy(feat_hbm.at[pl.ds(base, C), :], feat_v)
        gather.wait()

        def add_row(j, _):
            for k in range(D // LANES):
                sl = pl.ds(k * LANES, LANES)
                rows_v[j, sl] = rows_v[j, sl] + feat_v[j, sl]
            return 0

        lax.fori_loop(0, C, add_row, 0)
        pltpu.sync_copy(rows_v, out_hbm.at[pl.ds(base, C), :])
        return 0

    lax.fori_loop(0, N_CHUNKS, chunk, 0)


@jax.jit
def _gather_add(source_nodes, features, memory):
    mesh = plsc.VectorSubcoreMesh(core_axis_name="c", subcore_axis_name="s")
    f = pl.kernel(
        _sc_body,
        out_type=jax.ShapeDtypeStruct((B, D), jnp.float32),
        mesh=mesh,
        scratch_types=[
            pltpu.VMEM((C,), jnp.int32),
            pltpu.VMEM((C, D), jnp.float32),
            pltpu.VMEM((C, D), jnp.float32),
            pltpu.SemaphoreType.DMA,
        ],
    )
    return f(source_nodes, features, memory)


def kernel(source_nodes, source_node_raw_features, timestamps, n_layers,
           memory, time_W, time_b):
    idx = source_nodes.astype(jnp.int32)
    return _gather_add(idx, source_node_raw_features, memory)


# speedup vs baseline: 4.2306x; 1.9452x over previous
import jax
import jax.numpy as jnp
from jax import lax
from jax.experimental import pallas as pl
from jax.experimental.pallas import tpu as pltpu
from jax.experimental.pallas import tpu_sc as plsc

N_NODES = 100000
B = 625000
D = 128
LANES = 16

NC = 2
NS = 16
NW = NC * NS

C = 128
NBUF = 3

GROUPS = B // 8
GPW_BASE = GROUPS // NW
GPW_REM = GROUPS - GPW_BASE * NW
N_MAX = 8 * (GPW_BASE + 1)
N_CHUNKS = -(-N_MAX // C)
N_OUTER = N_CHUNKS // NBUF


def _sc_body(idx_hbm, feat_hbm, mem_hbm, out_hbm, *scratch):
    idx_v = scratch[0:NBUF]
    rows_v = scratch[NBUF:2 * NBUF]
    feat_v = scratch[2 * NBUF:3 * NBUF]
    sem_idx = scratch[3 * NBUF:4 * NBUF]
    sem_gat = scratch[4 * NBUF:5 * NBUF]
    sem_fea = scratch[5 * NBUF:6 * NBUF]
    sem_out = scratch[6 * NBUF:7 * NBUF]

    wid = lax.axis_index("s") * NC + lax.axis_index("c")
    extra = jnp.minimum(wid, GPW_REM)
    start = 8 * (wid * GPW_BASE + extra)
    n_rows = 8 * (GPW_BASE + jnp.where(wid < GPW_REM, 1, 0))

    def base(c):
        return start + jnp.minimum(c * C, n_rows - C)

    def fire_idx(c, b):
        pltpu.async_copy(idx_hbm.at[pl.ds(base(c), C)], idx_v[b], sem_idx[b])

    def fire_gather(b):
        pltpu.async_copy(mem_hbm.at[idx_v[b]], rows_v[b], sem_gat[b])

    def fire_feat(c, b):
        pltpu.async_copy(feat_hbm.at[pl.ds(base(c), C), :], feat_v[b],
                         sem_fea[b])

    def wait(src, dst, sem):
        pltpu.make_async_copy(src, dst, sem).wait()

    fire_idx(0, 0)
    wait(idx_hbm.at[pl.ds(base(0), C)], idx_v[0], sem_idx[0])
    fire_gather(0)
    fire_feat(0, 0)
    fire_idx(1, 1)

    def outer(g, _):
        for b in range(NBUF):
            c = g * NBUF + b
            bn = (b + 1) % NBUF

            @pl.when(c + 2 < N_CHUNKS)
            def _():
                fire_idx(c + 2, (b + 2) % NBUF)

            @pl.when(c + 1 < N_CHUNKS)
            def _():
                wait(idx_hbm.at[pl.ds(base(c + 1), C)], idx_v[bn],
                     sem_idx[bn])

                @pl.when(c >= 2)
                def _():
                    wait(rows_v[bn], out_hbm.at[pl.ds(base(c - 2), C), :],
                         sem_out[bn])

                fire_gather(bn)
                fire_feat(c + 1, bn)

            wait(mem_hbm.at[idx_v[b]], rows_v[b], sem_gat[b])
            wait(feat_hbm.at[pl.ds(base(c), C), :], feat_v[b], sem_fea[b])

            def add_row(j, _):
                for k in range(D // LANES):
                    sl = pl.ds(k * LANES, LANES)
                    rows_v[b][j, sl] = rows_v[b][j, sl] + feat_v[b][j, sl]
                return 0

            lax.fori_loop(0, C, add_row, 0)
            pltpu.async_copy(rows_v[b], out_hbm.at[pl.ds(base(c), C), :],
                             sem_out[b])
        return 0

    lax.fori_loop(0, N_OUTER, outer, 0)

    for b in range(NBUF):
        c = N_CHUNKS - NBUF + b
        wait(rows_v[b % NBUF], out_hbm.at[pl.ds(base(c), C), :],
             sem_out[c % NBUF])


@jax.jit
def _gather_add(source_nodes, features, memory):
    mesh = plsc.VectorSubcoreMesh(core_axis_name="c", subcore_axis_name="s")
    f = pl.kernel(
        _sc_body,
        out_type=jax.ShapeDtypeStruct((B, D), jnp.float32),
        mesh=mesh,
        scratch_types=(
            [pltpu.VMEM((C,), jnp.int32) for _ in range(NBUF)]
            + [pltpu.VMEM((C, D), jnp.float32) for _ in range(NBUF)]
            + [pltpu.VMEM((C, D), jnp.float32) for _ in range(NBUF)]
            + [pltpu.SemaphoreType.DMA for _ in range(4 * NBUF)]
        ),
    )
    return f(source_nodes, features, memory)


def kernel(source_nodes, source_node_raw_features, timestamps, n_layers,
           memory, time_W, time_b):
    idx = source_nodes.astype(jnp.int32)
    return _gather_add(idx, source_node_raw_features, memory)
